# baseline (speedup 1.0000x reference)
import jax
import jax.numpy as jnp
from jax import lax
from jax.experimental import pallas as pl
from jax.experimental.pallas import tpu as pltpu
from jax.experimental.pallas import tpu_sc as plsc

S = 2048
D = 4096
E = 64
C = 64
BS = 256
GRID = S // BS

NC = 2
NS = 16
EPC = E // NC
EPW = EPC // NS
KV = S // 16
TCH = S // NS


def _route_block(x_ref, w_ref, eidx_ref, gate_ref):
    x = x_ref[...]
    w = w_ref[...]
    lt = jax.lax.dot_general(
        w, x, (((1,), (1,)), ((), ())),
        preferred_element_type=jnp.float32)

    mx = jnp.max(lt, axis=0, keepdims=True)
    denom = jnp.sum(jnp.exp(lt - mx), axis=0, keepdims=True)
    gate_ref[...] = 1.0 / denom

    ismax = (lt == mx).astype(jnp.float32)
    er = jax.lax.broadcasted_iota(jnp.int32, (E, E), 0)
    ec = jax.lax.broadcasted_iota(jnp.int32, (E, E), 1)
    tri_e = (ec <= er).astype(jnp.float32)
    cummax = jax.lax.dot_general(
        tri_e, ismax, (((1,), (0,)), ((), ())),
        preferred_element_type=jnp.float32)
    mask = ismax * (cummax == 1.0)
    ei = jax.lax.broadcasted_iota(jnp.int32, (E, BS), 0)
    eidx_ref[...] = jnp.sum(
        jnp.where(mask != 0.0, ei, 0), axis=0, keepdims=True)


def _sc_rank_body(eidx_hbm, out_hbm, ids_v, part_v, red_v, shared):
    c = lax.axis_index("c")
    s = lax.axis_index("s")

    pltpu.sync_copy(eidx_hbm, ids_v)

    z16 = jnp.zeros((16,), jnp.int32)

    def zero_body(k, _):
        part_v[0, pl.ds(k * 16, 16)] = z16
        return 0

    lax.fori_loop(0, KV, zero_body, 0)

    lane = lax.iota(jnp.int32, 16)
    zi = jnp.zeros((16,), jnp.int32)
    for r in range(EPW):
        e = c * EPC + s * EPW + r

        def body(k, carry):
            ids16 = ids_v[0, pl.ds(k * 16, 16)]
            m = ids16 == e
            mi = jnp.where(m, 1, 0)
            pref = plsc.cumsum(mi)
            plsc.store_scatter(part_v, [zi, lane + k * 16], pref + carry,
                               mask=m)
            return carry + plsc.all_reduce_population_count(m)

        lax.fori_loop(0, KV, body, z16)

    pltpu.sync_copy(part_v, shared.at[s])
    plsc.subcore_barrier()

    pltpu.sync_copy(shared.at[:, :, pl.ds(s * TCH, TCH)], red_v)
    for j in range(TCH // 16):
        acc = z16
        for row in range(NS):
            acc = acc + red_v[row, 0, pl.ds(j * 16, 16)]
        part_v[0, pl.ds(j * 16, 16)] = acc
    pltpu.sync_copy(
        part_v.at[0, pl.ds(0, TCH)],
        out_hbm.at[c, pl.ds(s * TCH, TCH)])


def _sc_rank(eidx_row):
    mesh = plsc.VectorSubcoreMesh(core_axis_name="c", subcore_axis_name="s")
    fn = pl.kernel(
        _sc_rank_body,
        mesh=mesh,
        compiler_params=pltpu.CompilerParams(needs_layout_passes=False),
        out_type=jax.ShapeDtypeStruct((NC, S), jnp.int32),
        scratch_types=[
            pltpu.VMEM((1, S), jnp.int32),
            pltpu.VMEM((1, S), jnp.int32),
            pltpu.VMEM((NS, 1, TCH), jnp.int32),
            pltpu.VMEM_SHARED((NS, 1, S), jnp.int32),
        ],
    )
    return fn(eidx_row)


def _combine_block(eidx_ref, gate_ref, locp1_ref, cw_ref):
    eidx = eidx_ref[...].reshape(1, 1, BS)
    gate = gate_ref[...].reshape(1, 1, BS)
    lp = locp1_ref[...]
    loc = (lp[0:1, :] + lp[1:2, :] - 1).reshape(1, 1, BS)
    es = jax.lax.broadcasted_iota(jnp.int32, (E, 1, BS), 0)
    cs = jax.lax.broadcasted_iota(jnp.int32, (1, C, BS), 1)
    hit = (es == eidx) & (cs == loc)
    cw_ref[...] = jnp.where(hit, gate, 0.0)


def kernel(inp, W):
    x = inp.reshape(-1, inp.shape[-1])
    eidx, gate = pl.pallas_call(
        _route_block,
        grid=(GRID,),
        in_specs=[
            pl.BlockSpec((BS, D), lambda i: (i, 0)),
            pl.BlockSpec((E, D), lambda i: (0, 0)),
        ],
        out_specs=[
            pl.BlockSpec((1, BS), lambda i: (0, i)),
            pl.BlockSpec((1, BS), lambda i: (0, i)),
        ],
        out_shape=[
            jax.ShapeDtypeStruct((1, S), jnp.int32),
            jax.ShapeDtypeStruct((1, S), jnp.float32),
        ],
    )(x, W)

    locp1 = _sc_rank(eidx)

    cw_t = pl.pallas_call(
        _combine_block,
        grid=(GRID,),
        in_specs=[
            pl.BlockSpec((1, BS), lambda i: (0, i)),
            pl.BlockSpec((1, BS), lambda i: (0, i)),
            pl.BlockSpec((NC, BS), lambda i: (0, i)),
        ],
        out_specs=pl.BlockSpec((E, C, BS), lambda i: (0, 0, i)),
        out_shape=jax.ShapeDtypeStruct((E, C, S), jnp.float32),
    )(eidx, gate, locp1)

    cw = jnp.transpose(cw_t, (2, 0, 1))
    e0 = eidx.reshape(S, 1, 1)
    l0 = (locp1[0] + locp1[1]).reshape(S, 1, 1) - 1
    e_i = jax.lax.broadcasted_iota(jnp.int32, (S, E, C), 1)
    c_i = jax.lax.broadcasted_iota(jnp.int32, (S, E, C), 2)
    dm = (e_i == e0) & (c_i == l0)
    return cw, dm


# speedup vs baseline: 1.0289x; 1.0289x over previous
import jax
import jax.numpy as jnp
from jax import lax
from jax.experimental import pallas as pl
from jax.experimental.pallas import tpu as pltpu
from jax.experimental.pallas import tpu_sc as plsc

S = 2048
D = 4096
E = 64
C = 64
BS = 512
GRID = S // BS

NC = 2
NS = 16
EPC = E // NC
EPW = EPC // NS
KV = S // 16
TCH = S // NS


def _route_block(x_ref, w_ref, eidx_ref, gate_ref):
    x = x_ref[...]
    w = w_ref[...]
    lt = jax.lax.dot_general(
        w, x, (((1,), (1,)), ((), ())),
        preferred_element_type=jnp.float32)

    mx = jnp.max(lt, axis=0, keepdims=True)
    denom = jnp.sum(jnp.exp(lt - mx), axis=0, keepdims=True)
    gate_ref[...] = 1.0 / denom

    ismax = (lt == mx).astype(jnp.float32)
    er = jax.lax.broadcasted_iota(jnp.int32, (E, E), 0)
    ec = jax.lax.broadcasted_iota(jnp.int32, (E, E), 1)
    tri_e = (ec <= er).astype(jnp.float32)
    cummax = jax.lax.dot_general(
        tri_e, ismax, (((1,), (0,)), ((), ())),
        preferred_element_type=jnp.float32)
    mask = ismax * (cummax == 1.0)
    ei = jax.lax.broadcasted_iota(jnp.int32, (E, BS), 0)
    eidx_ref[...] = jnp.sum(
        jnp.where(mask != 0.0, ei, 0), axis=0, keepdims=True)


def _sc_rank_body(eidx_hbm, out_hbm, ids_v, part_v, red_v, shared):
    c = lax.axis_index("c")
    s = lax.axis_index("s")

    pltpu.sync_copy(eidx_hbm, ids_v)

    z16 = jnp.zeros((16,), jnp.int32)

    def zero_body(k, _):
        part_v[0, pl.ds(k * 16, 16)] = z16
        return 0

    lax.fori_loop(0, KV, zero_body, 0)

    lane = lax.iota(jnp.int32, 16)
    zi = jnp.zeros((16,), jnp.int32)
    for r in range(EPW):
        e = c * EPC + s * EPW + r

        def body(k, carry):
            ids16 = ids_v[0, pl.ds(k * 16, 16)]
            m = ids16 == e
            mi = jnp.where(m, 1, 0)
            pref = plsc.cumsum(mi)
            plsc.store_scatter(part_v, [zi, lane + k * 16], pref + carry,
                               mask=m)
            return carry + plsc.all_reduce_population_count(m)

        lax.fori_loop(0, KV, body, z16)

    pltpu.sync_copy(part_v, shared.at[s])
    plsc.subcore_barrier()

    pltpu.sync_copy(shared.at[:, :, pl.ds(s * TCH, TCH)], red_v)
    for j in range(TCH // 16):
        acc = z16
        for row in range(NS):
            acc = acc + red_v[row, 0, pl.ds(j * 16, 16)]
        part_v[0, pl.ds(j * 16, 16)] = acc
    pltpu.sync_copy(
        part_v.at[0, pl.ds(0, TCH)],
        out_hbm.at[c, pl.ds(s * TCH, TCH)])


def _sc_rank(eidx_row):
    mesh = plsc.VectorSubcoreMesh(core_axis_name="c", subcore_axis_name="s")
    fn = pl.kernel(
        _sc_rank_body,
        mesh=mesh,
        compiler_params=pltpu.CompilerParams(needs_layout_passes=False),
        out_type=jax.ShapeDtypeStruct((NC, S), jnp.int32),
        scratch_types=[
            pltpu.VMEM((1, S), jnp.int32),
            pltpu.VMEM((1, S), jnp.int32),
            pltpu.VMEM((NS, 1, TCH), jnp.int32),
            pltpu.VMEM_SHARED((NS, 1, S), jnp.int32),
        ],
    )
    return fn(eidx_row)


def _combine_block(eidx_ref, gate_ref, locp1_ref, cw_ref):
    eidx = eidx_ref[...].reshape(1, 1, BS)
    gate = gate_ref[...].reshape(1, 1, BS)
    lp = locp1_ref[...]
    loc = (lp[0:1, :] + lp[1:2, :] - 1).reshape(1, 1, BS)
    es = jax.lax.broadcasted_iota(jnp.int32, (E, 1, BS), 0)
    cs = jax.lax.broadcasted_iota(jnp.int32, (1, C, BS), 1)
    hit = (es == eidx) & (cs == loc)
    cw_ref[...] = jnp.where(hit, gate, 0.0)


def kernel(inp, W):
    x = inp.reshape(-1, inp.shape[-1])
    eidx, gate = pl.pallas_call(
        _route_block,
        grid=(GRID,),
        in_specs=[
            pl.BlockSpec((BS, D), lambda i: (i, 0)),
            pl.BlockSpec((E, D), lambda i: (0, 0)),
        ],
        out_specs=[
            pl.BlockSpec((1, BS), lambda i: (0, i)),
            pl.BlockSpec((1, BS), lambda i: (0, i)),
        ],
        out_shape=[
            jax.ShapeDtypeStruct((1, S), jnp.int32),
            jax.ShapeDtypeStruct((1, S), jnp.float32),
        ],
    )(x, W)

    locp1 = _sc_rank(eidx)

    cw_t = pl.pallas_call(
        _combine_block,
        grid=(GRID,),
        in_specs=[
            pl.BlockSpec((1, BS), lambda i: (0, i)),
            pl.BlockSpec((1, BS), lambda i: (0, i)),
            pl.BlockSpec((NC, BS), lambda i: (0, i)),
        ],
        out_specs=pl.BlockSpec((E, C, BS), lambda i: (0, 0, i)),
        out_shape=jax.ShapeDtypeStruct((E, C, S), jnp.float32),
    )(eidx, gate, locp1)

    cw = jnp.transpose(cw_t, (2, 0, 1))
    e0 = eidx.reshape(S, 1, 1)
    l0 = (locp1[0] + locp1[1]).reshape(S, 1, 1) - 1
    e_i = jax.lax.broadcasted_iota(jnp.int32, (S, E, C), 1)
    c_i = jax.lax.broadcasted_iota(jnp.int32, (S, E, C), 2)
    dm = (e_i == e0) & (c_i == l0)
    return cw, dm
